# baseline (speedup 1.0000x reference)
import functools

import jax
import jax.numpy as jnp
from jax import lax
from jax.experimental import pallas as pl
from jax.experimental.pallas import tpu as pltpu

_A = 0.88
_B = 0.88
_S = 64.0

_ACOS_COEFFS = (
    -0.0012624911,
    0.0066700901,
    -0.0170881256,
    0.0308918810,
    -0.0501743046,
    0.0889789874,
    -0.2145988016,
    1.5707963050,
)


def _margin_from_target(t):
    t = jnp.minimum(jnp.maximum(t, 0.0), 1.0)
    y = jnp.maximum(1.0 - t, 1e-12)
    sqrt_y = y * lax.rsqrt(y)
    p = jnp.full_like(t, _ACOS_COEFFS[0])
    for c in _ACOS_COEFFS[1:]:
        p = p * t + c
    acos_t = sqrt_y * p
    return ((-_A) * acos_t + _B) * _S


def _tc_merge_body(lbl_ref, xt_ref, out_ref, *, bc):
    j = pl.program_id(0)
    x = xt_ref[...]
    lbl = lbl_ref[...]
    row = lax.broadcasted_iota(jnp.int32, x.shape, 0) + j * bc
    mask = row == lbl
    t = jnp.sum(jnp.where(mask, x, 0.0), axis=0, keepdims=True)
    nv = _margin_from_target(t)
    out_ref[...] = jnp.where(mask, nv, x * _S)


_BCR = 400
_K = 12


def _tc_ring_body(lbl_ref, xt_ref, out_ref, in_buf, out_buf, in_sems,
                  out_sems, *, n):
    bcr = _BCR
    lbl = lbl_ref[...]

    def read_chunk(i, slot):
        pltpu.make_async_copy(
            xt_ref.at[pl.ds(i * bcr, bcr)], in_buf.at[slot], in_sems.at[slot]
        ).start()

    def write_chunk(i, slot):
        return pltpu.make_async_copy(
            out_buf.at[slot], out_ref.at[pl.ds(i * bcr, bcr)],
            out_sems.at[slot]
        )

    for k in range(_K):
        read_chunk(k, k)

    def step(i, carry):
        slot = lax.rem(i, _K)
        base = pl.multiple_of(i * bcr, bcr)
        pltpu.make_async_copy(
            xt_ref.at[pl.ds(base, bcr)], in_buf.at[slot], in_sems.at[slot]
        ).wait()

        @pl.when(i >= _K)
        def _():
            write_chunk(i - _K, slot).wait()

        x = in_buf[slot]
        row = lax.broadcasted_iota(jnp.int32, x.shape, 0) + i * bcr
        mask = row == lbl
        t = jnp.sum(jnp.where(mask, x, 0.0), axis=0, keepdims=True)
        nv = _margin_from_target(t)
        out_buf[slot] = jnp.where(mask, nv, x * _S)
        write_chunk(i, slot).start()

        @pl.when(i + _K < n)
        def _():
            read_chunk(i + _K, slot)

        return carry

    lax.fori_loop(0, n, step, 0)
    for k in range(_K):
        i = n - _K + k
        write_chunk(i, i % _K).wait()


def _ring_kernel(cosine, label):
    B, C = cosine.shape
    xt = jnp.swapaxes(cosine, 0, 1)
    n = C // _BCR
    out_t = pl.pallas_call(
        functools.partial(_tc_ring_body, n=n),
        out_shape=jax.ShapeDtypeStruct((C, B), jnp.float32),
        in_specs=[
            pl.BlockSpec(memory_space=pltpu.VMEM),
            pl.BlockSpec(memory_space=pl.ANY),
        ],
        out_specs=pl.BlockSpec(memory_space=pl.ANY),
        scratch_shapes=[
            pltpu.VMEM((_K, _BCR, B), jnp.float32),
            pltpu.VMEM((_K, _BCR, B), jnp.float32),
            pltpu.SemaphoreType.DMA((_K,)),
            pltpu.SemaphoreType.DMA((_K,)),
        ],
        name="tc_ring_merge",
    )(label.reshape(1, B), xt)
    return jnp.swapaxes(out_t, 0, 1)


def kernel(cosine, label):
    return _ring_kernel(cosine, label)
    B, C = cosine.shape
    xt = jnp.swapaxes(cosine, 0, 1)
    bc = 3072
    out_t = pl.pallas_call(
        functools.partial(_tc_merge_body, bc=bc),
        out_shape=jax.ShapeDtypeStruct((C, B), jnp.float32),
        grid=(pl.cdiv(C, bc),),
        in_specs=[
            pl.BlockSpec((1, B), lambda j: (0, 0)),
            pl.BlockSpec((bc, B), lambda j: (j, 0)),
        ],
        out_specs=pl.BlockSpec((bc, B), lambda j: (j, 0)),
        compiler_params=pltpu.CompilerParams(
            dimension_semantics=("arbitrary",),
        ),
        name="tc_scale_merge",
    )(label.reshape(1, B), xt)
    return jnp.swapaxes(out_t, 0, 1)


# speedup vs baseline: 1.0016x; 1.0016x over previous
import functools

import jax
import jax.numpy as jnp
from jax import lax
from jax.experimental import pallas as pl
from jax.experimental.pallas import tpu as pltpu

_A = 0.88
_B = 0.88
_S = 64.0

_ACOS_COEFFS = (
    -0.0012624911,
    0.0066700901,
    -0.0170881256,
    0.0308918810,
    -0.0501743046,
    0.0889789874,
    -0.2145988016,
    1.5707963050,
)


def _margin_from_target(t):
    t = jnp.minimum(jnp.maximum(t, 0.0), 1.0)
    y = jnp.maximum(1.0 - t, 1e-12)
    sqrt_y = y * lax.rsqrt(y)
    p = jnp.full_like(t, _ACOS_COEFFS[0])
    for c in _ACOS_COEFFS[1:]:
        p = p * t + c
    acos_t = sqrt_y * p
    return ((-_A) * acos_t + _B) * _S


def _tc_merge_body(lbl_ref, xt_ref, out_ref, *, bc):
    j = pl.program_id(0)
    x = xt_ref[...]
    lbl = lbl_ref[...]
    row = lax.broadcasted_iota(jnp.int32, x.shape, 0) + j * bc
    mask = row == lbl
    t = jnp.sum(jnp.where(mask, x, 0.0), axis=0, keepdims=True)
    nv = _margin_from_target(t)
    out_ref[...] = jnp.where(mask, nv, x * _S)


_BCR = 1000
_K = 6


def _tc_ring_body(lbl_ref, xt_ref, out_ref, in_buf, out_buf, in_sems,
                  out_sems, *, n):
    bcr = _BCR
    lbl = lbl_ref[...]

    def read_chunk(i, slot):
        pltpu.make_async_copy(
            xt_ref.at[pl.ds(i * bcr, bcr)], in_buf.at[slot], in_sems.at[slot]
        ).start()

    def write_chunk(i, slot):
        return pltpu.make_async_copy(
            out_buf.at[slot], out_ref.at[pl.ds(i * bcr, bcr)],
            out_sems.at[slot]
        )

    for k in range(_K):
        read_chunk(k, k)

    def step(i, carry):
        slot = lax.rem(i, _K)
        base = pl.multiple_of(i * bcr, bcr)
        pltpu.make_async_copy(
            xt_ref.at[pl.ds(base, bcr)], in_buf.at[slot], in_sems.at[slot]
        ).wait()

        @pl.when(i >= _K)
        def _():
            write_chunk(i - _K, slot).wait()

        x = in_buf[slot]
        row = lax.broadcasted_iota(jnp.int32, x.shape, 0) + i * bcr
        mask = row == lbl
        t = jnp.sum(jnp.where(mask, x, 0.0), axis=0, keepdims=True)
        nv = _margin_from_target(t)
        out_buf[slot] = jnp.where(mask, nv, x * _S)
        write_chunk(i, slot).start()

        @pl.when(i + _K < n)
        def _():
            read_chunk(i + _K, slot)

        return carry

    lax.fori_loop(0, n, step, 0)
    for k in range(_K):
        i = n - _K + k
        write_chunk(i, i % _K).wait()


def _ring_kernel(cosine, label):
    B, C = cosine.shape
    xt = jnp.swapaxes(cosine, 0, 1)
    n = C // _BCR
    out_t = pl.pallas_call(
        functools.partial(_tc_ring_body, n=n),
        out_shape=jax.ShapeDtypeStruct((C, B), jnp.float32),
        in_specs=[
            pl.BlockSpec(memory_space=pltpu.VMEM),
            pl.BlockSpec(memory_space=pl.ANY),
        ],
        out_specs=pl.BlockSpec(memory_space=pl.ANY),
        scratch_shapes=[
            pltpu.VMEM((_K, _BCR, B), jnp.float32),
            pltpu.VMEM((_K, _BCR, B), jnp.float32),
            pltpu.SemaphoreType.DMA((_K,)),
            pltpu.SemaphoreType.DMA((_K,)),
        ],
        name="tc_ring_merge",
    )(label.reshape(1, B), xt)
    return jnp.swapaxes(out_t, 0, 1)


def kernel(cosine, label):
    return _ring_kernel(cosine, label)
    B, C = cosine.shape
    xt = jnp.swapaxes(cosine, 0, 1)
    bc = 3072
    out_t = pl.pallas_call(
        functools.partial(_tc_merge_body, bc=bc),
        out_shape=jax.ShapeDtypeStruct((C, B), jnp.float32),
        grid=(pl.cdiv(C, bc),),
        in_specs=[
            pl.BlockSpec((1, B), lambda j: (0, 0)),
            pl.BlockSpec((bc, B), lambda j: (j, 0)),
        ],
        out_specs=pl.BlockSpec((bc, B), lambda j: (j, 0)),
        compiler_params=pltpu.CompilerParams(
            dimension_semantics=("arbitrary",),
        ),
        name="tc_scale_merge",
    )(label.reshape(1, B), xt)
    return jnp.swapaxes(out_t, 0, 1)
